# Initial kernel scaffold; baseline (speedup 1.0000x reference)
import jax
import jax.numpy as jnp
from jax.experimental import pallas as pl


def kernel(x, train_pos_edge_index, W1_out, b1_out, W1_root, W2_out, b2_out, W2_root):
    raise NotImplementedError("write your pallas kernel here")


# speedup vs baseline: 20.9810x; 20.9810x over previous
import functools

import jax
import jax.numpy as jnp
from jax import lax
from jax.experimental import pallas as pl
from jax.experimental.pallas import tpu as pltpu
from jax.experimental.pallas import tpu_sc as plsc

N = 10000
E = 320000
DIN = 128
DHID = 128
DOUT = 16
DH = DHID // 2

NC = 2
NS = 16
NW = NC * NS
CH = 128
NCHUNK1 = 157
NCHUNK2 = 79
NPAD = 10112
RPT = NPAD // NS

BM = 2000


def _sc_l1_body(feat_hbm, rows_hbm, cols_hbm, zf_hbm, zd_hbm, ones_hbm,
                pa_hbm, pd_hbm,
                row_v, col_v, gbuf, ones_v, acc_sh, deg_sh, sem):
  cid = lax.axis_index("c")
  sid = lax.axis_index("s")
  sl = pl.ds(sid * RPT, RPT)

  pltpu.sync_copy(zf_hbm, acc_sh.at[sl])
  pltpu.sync_copy(zd_hbm, deg_sh.at[sl])
  pltpu.sync_copy(ones_hbm, ones_v)
  pltpu.sync_copy(rows_hbm.at[cid, sid], row_v)
  pltpu.sync_copy(cols_hbm.at[sid], col_v)
  plsc.subcore_barrier()

  def chunk(j, carry):
    pltpu.async_copy(feat_hbm.at[row_v.at[j]], gbuf, sem).wait()
    pltpu.sync_copy(gbuf, acc_sh.at[col_v.at[j]], add=True)

    @pl.when(lax.rem(j, 2) == cid)
    def _():
      pltpu.sync_copy(ones_v, deg_sh.at[col_v.at[j]], add=True)

    return carry

  lax.fori_loop(0, NCHUNK1, chunk, 0)
  plsc.subcore_barrier()

  pltpu.sync_copy(acc_sh.at[sl], pa_hbm.at[cid, sl])
  pltpu.sync_copy(deg_sh.at[sl], pd_hbm.at[cid, sl])


_sc_l1 = functools.partial(
    pl.kernel,
    out_type=[jax.ShapeDtypeStruct((NC, NPAD, DH), jnp.float32),
              jax.ShapeDtypeStruct((NC, NPAD, 16), jnp.float32)],
    mesh=plsc.VectorSubcoreMesh(core_axis_name="c", subcore_axis_name="s"),
    scratch_types=[
        pltpu.VMEM((NCHUNK1, CH), jnp.int32),
        pltpu.VMEM((NCHUNK1, CH), jnp.int32),
        pltpu.VMEM((CH, DH), jnp.float32),
        pltpu.VMEM((CH, 16), jnp.float32),
        pltpu.VMEM_SHARED((NPAD, DH), jnp.float32),
        pltpu.VMEM_SHARED((NPAD, 16), jnp.float32),
        pltpu.SemaphoreType.DMA,
    ],
    compiler_params=pltpu.CompilerParams(use_tc_tiling_on_sc=False),
    )(_sc_l1_body)


def _sc_l2_body(feat_hbm, rows_hbm, cols_hbm, zf_hbm,
                pa_hbm,
                row_v, col_v, gbuf, acc_sh, sem):
  cid = lax.axis_index("c")
  sid = lax.axis_index("s")
  wid = cid * NS + sid
  sl = pl.ds(sid * RPT, RPT)

  pltpu.sync_copy(zf_hbm, acc_sh.at[sl])
  pltpu.sync_copy(rows_hbm.at[wid], row_v)
  pltpu.sync_copy(cols_hbm.at[wid], col_v)
  plsc.subcore_barrier()

  def chunk(j, carry):
    pltpu.async_copy(feat_hbm.at[row_v.at[j]], gbuf, sem).wait()
    pltpu.sync_copy(gbuf, acc_sh.at[col_v.at[j]], add=True)
    return carry

  lax.fori_loop(0, NCHUNK2, chunk, 0)
  plsc.subcore_barrier()

  pltpu.sync_copy(acc_sh.at[sl], pa_hbm.at[cid, sl])


_sc_l2 = functools.partial(
    pl.kernel,
    out_type=[jax.ShapeDtypeStruct((NC, NPAD, DOUT), jnp.float32)],
    mesh=plsc.VectorSubcoreMesh(core_axis_name="c", subcore_axis_name="s"),
    scratch_types=[
        pltpu.VMEM((NCHUNK2, CH), jnp.int32),
        pltpu.VMEM((NCHUNK2, CH), jnp.int32),
        pltpu.VMEM((CH, DOUT), jnp.float32),
        pltpu.VMEM_SHARED((NPAD, DOUT), jnp.float32),
        pltpu.SemaphoreType.DMA,
    ],
    compiler_params=pltpu.CompilerParams(use_tc_tiling_on_sc=False),
    )(_sc_l2_body)


def _phase_a(x_ref, wo_ref, wr_ref, b_ref, y2_ref, r_ref):
  xb = x_ref[...]
  y = jnp.dot(xb, wo_ref[...], preferred_element_type=jnp.float32)
  y2_ref[0] = y[:, :DH]
  y2_ref[1] = y[:, DH:]
  r_ref[...] = (jnp.dot(xb, wr_ref[...], preferred_element_type=jnp.float32)
                + b_ref[...])


def _phase_c(y0_ref, y1_ref, a0_ref, a1_ref, d0_ref, d1_ref, r1_ref, wo_ref,
             wr_ref, b_ref, p_ref, r2_ref, dinv_ref):
  deg = 1.0 + d0_ref[:, :1] + d1_ref[:, :1]
  dinv = 1.0 / jnp.maximum(deg, 1.0)
  agg = jnp.concatenate(
      [y0_ref[...] + a0_ref[...], y1_ref[...] + a1_ref[...]], axis=1) * dinv
  h = jnp.maximum(agg + r1_ref[...], 0.0)
  p_ref[...] = jnp.dot(h, wo_ref[...], preferred_element_type=jnp.float32)
  r2_ref[...] = (jnp.dot(h, wr_ref[...], preferred_element_type=jnp.float32)
                 + b_ref[...])
  dinv_ref[...] = jnp.broadcast_to(dinv, dinv_ref.shape)


def _phase_e(p_ref, q0_ref, q1_ref, dinv_ref, r2_ref, o_ref):
  o_ref[...] = ((p_ref[...] + q0_ref[...] + q1_ref[...]) * dinv_ref[...]
                + r2_ref[...])


def kernel(x, train_pos_edge_index, W1_out, b1_out, W1_root, W2_out, b2_out,
           W2_root):
  row = train_pos_edge_index[0]
  col = train_pos_edge_index[1]
  colm = jnp.where(row == col, jnp.int32(N), col)

  pad1 = NS * NCHUNK1 * CH - E
  rows1 = jnp.concatenate(
      [row, jnp.zeros((pad1,), jnp.int32)]).reshape(NS, NCHUNK1, CH)
  rows1 = jnp.stack([rows1, rows1 + N])
  cols1 = jnp.concatenate(
      [colm, jnp.full((pad1,), N, jnp.int32)]).reshape(NS, NCHUNK1, CH)

  pad2 = NW * NCHUNK2 * CH - E
  rows2 = jnp.concatenate(
      [row, jnp.zeros((pad2,), jnp.int32)]).reshape(NW, NCHUNK2, CH)
  cols2 = jnp.concatenate(
      [colm, jnp.full((pad2,), N, jnp.int32)]).reshape(NW, NCHUNK2, CH)

  zf = jnp.zeros((RPT, DH), jnp.float32)
  zd = jnp.zeros((RPT, 16), jnp.float32)
  ones = jnp.ones((CH, 16), jnp.float32)

  grid = (N // BM,)
  full = lambda shape: pl.BlockSpec(shape, lambda i: (0,) * len(shape))
  rows_spec = lambda width: pl.BlockSpec((BM, width), lambda i: (i, 0))

  y2, r1 = pl.pallas_call(
      _phase_a,
      grid=grid,
      in_specs=[rows_spec(DIN), full((DIN, DHID)), full((DIN, DHID)),
                full((1, DHID))],
      out_specs=[pl.BlockSpec((NC, BM, DH), lambda i: (0, i, 0)),
                 rows_spec(DHID)],
      out_shape=[jax.ShapeDtypeStruct((NC, N, DH), jnp.float32),
                 jax.ShapeDtypeStruct((N, DHID), jnp.float32)],
  )(x, W1_out, W1_root, b1_out.reshape(1, DHID))

  pa, pd = _sc_l1(y2.reshape(NC * N, DH), rows1, cols1, zf, zd, ones)

  p, r2, dinv = pl.pallas_call(
      _phase_c,
      grid=grid,
      in_specs=[rows_spec(DH), rows_spec(DH), rows_spec(DH), rows_spec(DH),
                rows_spec(16), rows_spec(16), rows_spec(DHID),
                full((DHID, DOUT)), full((DHID, DOUT)),
                pl.BlockSpec((1, DOUT), lambda i: (0, 0))],
      out_specs=[rows_spec(DOUT), rows_spec(DOUT), rows_spec(16)],
      out_shape=[jax.ShapeDtypeStruct((N, DOUT), jnp.float32),
                 jax.ShapeDtypeStruct((N, DOUT), jnp.float32),
                 jax.ShapeDtypeStruct((N, 16), jnp.float32)],
  )(y2[0], y2[1], pa[0, :N], pa[1, :N], pd[0, :N], pd[1, :N], r1,
    W2_out, W2_root, b2_out.reshape(1, DOUT))

  (pa2,) = _sc_l2(p, rows2, cols2, zd[:, :DOUT])

  out = pl.pallas_call(
      _phase_e,
      grid=grid,
      in_specs=[rows_spec(DOUT), rows_spec(DOUT), rows_spec(DOUT),
                rows_spec(16), rows_spec(DOUT)],
      out_specs=rows_spec(DOUT),
      out_shape=jax.ShapeDtypeStruct((N, DOUT), jnp.float32),
  )(p, pa2[0, :N], pa2[1, :N], dinv, r2)
  return out


# speedup vs baseline: 24.1430x; 1.1507x over previous
import functools

import jax
import jax.numpy as jnp
from jax import lax
from jax.experimental import pallas as pl
from jax.experimental.pallas import tpu as pltpu
from jax.experimental.pallas import tpu_sc as plsc

N = 10000
E = 320000
DIN = 128
DHID = 128
DOUT = 16
DH = DHID // 2

NC = 2
NS = 16
NW = NC * NS
CH = 128
NCHUNK1 = 158
NCHUNK2 = 80
NPAD = 10112
RPT = NPAD // NS

BM = 2000


def _sc_l1_body(feat_hbm, rows_hbm, cols_hbm, zf_hbm, zd_hbm, ones_hbm,
                pa_hbm, pd_hbm,
                row_v, col_v, g0, g1, ones_v, acc_sh, deg_sh, sem0, sem1):
  cid = lax.axis_index("c")
  sid = lax.axis_index("s")
  sl = pl.ds(sid * RPT, RPT)

  pltpu.sync_copy(zf_hbm, acc_sh.at[sl])
  pltpu.sync_copy(zd_hbm, deg_sh.at[sl])
  pltpu.sync_copy(ones_hbm, ones_v)
  pltpu.sync_copy(rows_hbm.at[cid, sid], row_v)
  pltpu.sync_copy(cols_hbm.at[sid], col_v)
  plsc.subcore_barrier()

  nh = NCHUNK1 // 2
  pltpu.async_copy(feat_hbm.at[row_v.at[0]], g0, sem0)
  pltpu.async_copy(feat_hbm.at[row_v.at[1]], g1, sem1)

  def pair(i, carry):
    j0 = i * 2
    j1 = j0 + 1
    pltpu.make_async_copy(feat_hbm.at[row_v.at[j0]], g0, sem0).wait()
    pltpu.sync_copy(g0, acc_sh.at[col_v.at[j0]], add=True)

    @pl.when(cid == 0)
    def _():
      pltpu.sync_copy(ones_v, deg_sh.at[col_v.at[j0]], add=True)

    @pl.when(i + 1 < nh)
    def _():
      pltpu.async_copy(feat_hbm.at[row_v.at[j0 + 2]], g0, sem0)

    pltpu.make_async_copy(feat_hbm.at[row_v.at[j1]], g1, sem1).wait()
    pltpu.sync_copy(g1, acc_sh.at[col_v.at[j1]], add=True)

    @pl.when(cid == 1)
    def _():
      pltpu.sync_copy(ones_v, deg_sh.at[col_v.at[j1]], add=True)

    @pl.when(i + 1 < nh)
    def _():
      pltpu.async_copy(feat_hbm.at[row_v.at[j1 + 2]], g1, sem1)

    return carry

  lax.fori_loop(0, nh, pair, 0)
  plsc.subcore_barrier()

  pltpu.sync_copy(acc_sh.at[sl], pa_hbm.at[cid, sl])
  pltpu.sync_copy(deg_sh.at[sl], pd_hbm.at[cid, sl])


_sc_l1 = functools.partial(
    pl.kernel,
    out_type=[jax.ShapeDtypeStruct((NC, NPAD, DH), jnp.float32),
              jax.ShapeDtypeStruct((NC, NPAD, 16), jnp.float32)],
    mesh=plsc.VectorSubcoreMesh(core_axis_name="c", subcore_axis_name="s"),
    scratch_types=[
        pltpu.VMEM((NCHUNK1, CH), jnp.int32),
        pltpu.VMEM((NCHUNK1, CH), jnp.int32),
        pltpu.VMEM((CH, DH), jnp.float32),
        pltpu.VMEM((CH, DH), jnp.float32),
        pltpu.VMEM((CH, 16), jnp.float32),
        pltpu.VMEM_SHARED((NPAD, DH), jnp.float32),
        pltpu.VMEM_SHARED((NPAD, 16), jnp.float32),
        pltpu.SemaphoreType.DMA,
        pltpu.SemaphoreType.DMA,
    ],
    compiler_params=pltpu.CompilerParams(use_tc_tiling_on_sc=False),
    )(_sc_l1_body)


def _sc_l2_body(feat_hbm, rows_hbm, cols_hbm, zf_hbm,
                pa_hbm,
                row_v, col_v, g0, g1, acc_sh, sem0, sem1):
  cid = lax.axis_index("c")
  sid = lax.axis_index("s")
  wid = cid * NS + sid
  sl = pl.ds(sid * RPT, RPT)

  pltpu.sync_copy(zf_hbm, acc_sh.at[sl])
  pltpu.sync_copy(rows_hbm.at[wid], row_v)
  pltpu.sync_copy(cols_hbm.at[wid], col_v)
  plsc.subcore_barrier()

  nh = NCHUNK2 // 2
  pltpu.async_copy(feat_hbm.at[row_v.at[0]], g0, sem0)
  pltpu.async_copy(feat_hbm.at[row_v.at[1]], g1, sem1)

  def pair(i, carry):
    j0 = i * 2
    j1 = j0 + 1
    pltpu.make_async_copy(feat_hbm.at[row_v.at[j0]], g0, sem0).wait()
    pltpu.sync_copy(g0, acc_sh.at[col_v.at[j0]], add=True)

    @pl.when(i + 1 < nh)
    def _():
      pltpu.async_copy(feat_hbm.at[row_v.at[j0 + 2]], g0, sem0)

    pltpu.make_async_copy(feat_hbm.at[row_v.at[j1]], g1, sem1).wait()
    pltpu.sync_copy(g1, acc_sh.at[col_v.at[j1]], add=True)

    @pl.when(i + 1 < nh)
    def _():
      pltpu.async_copy(feat_hbm.at[row_v.at[j1 + 2]], g1, sem1)

    return carry

  lax.fori_loop(0, nh, pair, 0)
  plsc.subcore_barrier()

  pltpu.sync_copy(acc_sh.at[sl], pa_hbm.at[cid, sl])


_sc_l2 = functools.partial(
    pl.kernel,
    out_type=[jax.ShapeDtypeStruct((NC, NPAD, DOUT), jnp.float32)],
    mesh=plsc.VectorSubcoreMesh(core_axis_name="c", subcore_axis_name="s"),
    scratch_types=[
        pltpu.VMEM((NCHUNK2, CH), jnp.int32),
        pltpu.VMEM((NCHUNK2, CH), jnp.int32),
        pltpu.VMEM((CH, DOUT), jnp.float32),
        pltpu.VMEM((CH, DOUT), jnp.float32),
        pltpu.VMEM_SHARED((NPAD, DOUT), jnp.float32),
        pltpu.SemaphoreType.DMA,
        pltpu.SemaphoreType.DMA,
    ],
    compiler_params=pltpu.CompilerParams(use_tc_tiling_on_sc=False),
    )(_sc_l2_body)


def _phase_a(x_ref, wo_ref, wr_ref, b_ref, y2_ref, r_ref):
  xb = x_ref[...]
  y = jnp.dot(xb, wo_ref[...], preferred_element_type=jnp.float32)
  y2_ref[0] = y[:, :DH]
  y2_ref[1] = y[:, DH:]
  r_ref[...] = (jnp.dot(xb, wr_ref[...], preferred_element_type=jnp.float32)
                + b_ref[...])


def _phase_c(y0_ref, y1_ref, a0_ref, a1_ref, d0_ref, d1_ref, r1_ref, wo_ref,
             wr_ref, b_ref, p_ref, r2_ref, dinv_ref):
  deg = 1.0 + d0_ref[:, :1] + d1_ref[:, :1]
  dinv = 1.0 / jnp.maximum(deg, 1.0)
  agg = jnp.concatenate(
      [y0_ref[...] + a0_ref[...], y1_ref[...] + a1_ref[...]], axis=1) * dinv
  h = jnp.maximum(agg + r1_ref[...], 0.0)
  p_ref[...] = jnp.dot(h, wo_ref[...], preferred_element_type=jnp.float32)
  r2_ref[...] = (jnp.dot(h, wr_ref[...], preferred_element_type=jnp.float32)
                 + b_ref[...])
  dinv_ref[...] = jnp.broadcast_to(dinv, dinv_ref.shape)


def _phase_e(p_ref, q0_ref, q1_ref, dinv_ref, r2_ref, o_ref):
  o_ref[...] = ((p_ref[...] + q0_ref[...] + q1_ref[...]) * dinv_ref[...]
                + r2_ref[...])


def kernel(x, train_pos_edge_index, W1_out, b1_out, W1_root, W2_out, b2_out,
           W2_root):
  row = train_pos_edge_index[0]
  col = train_pos_edge_index[1]
  colm = jnp.where(row == col, jnp.int32(N), col)

  pad1 = NS * NCHUNK1 * CH - E
  rows1 = jnp.concatenate(
      [row, jnp.zeros((pad1,), jnp.int32)]).reshape(NS, NCHUNK1, CH)
  rows1 = jnp.stack([rows1, rows1 + N])
  cols1 = jnp.concatenate(
      [colm, jnp.full((pad1,), N, jnp.int32)]).reshape(NS, NCHUNK1, CH)

  pad2 = NW * NCHUNK2 * CH - E
  rows2 = jnp.concatenate(
      [row, jnp.zeros((pad2,), jnp.int32)]).reshape(NW, NCHUNK2, CH)
  cols2 = jnp.concatenate(
      [colm, jnp.full((pad2,), N, jnp.int32)]).reshape(NW, NCHUNK2, CH)

  zf = jnp.zeros((RPT, DH), jnp.float32)
  zd = jnp.zeros((RPT, 16), jnp.float32)
  ones = jnp.ones((CH, 16), jnp.float32)

  grid = (N // BM,)
  full = lambda shape: pl.BlockSpec(shape, lambda i: (0,) * len(shape))
  rows_spec = lambda width: pl.BlockSpec((BM, width), lambda i: (i, 0))

  y2, r1 = pl.pallas_call(
      _phase_a,
      grid=grid,
      in_specs=[rows_spec(DIN), full((DIN, DHID)), full((DIN, DHID)),
                full((1, DHID))],
      out_specs=[pl.BlockSpec((NC, BM, DH), lambda i: (0, i, 0)),
                 rows_spec(DHID)],
      out_shape=[jax.ShapeDtypeStruct((NC, N, DH), jnp.float32),
                 jax.ShapeDtypeStruct((N, DHID), jnp.float32)],
  )(x, W1_out, W1_root, b1_out.reshape(1, DHID))

  pa, pd = _sc_l1(y2.reshape(NC * N, DH), rows1, cols1, zf, zd, ones)

  p, r2, dinv = pl.pallas_call(
      _phase_c,
      grid=grid,
      in_specs=[rows_spec(DH), rows_spec(DH), rows_spec(DH), rows_spec(DH),
                rows_spec(16), rows_spec(16), rows_spec(DHID),
                full((DHID, DOUT)), full((DHID, DOUT)),
                pl.BlockSpec((1, DOUT), lambda i: (0, 0))],
      out_specs=[rows_spec(DOUT), rows_spec(DOUT), rows_spec(16)],
      out_shape=[jax.ShapeDtypeStruct((N, DOUT), jnp.float32),
                 jax.ShapeDtypeStruct((N, DOUT), jnp.float32),
                 jax.ShapeDtypeStruct((N, 16), jnp.float32)],
  )(y2[0], y2[1], pa[0, :N], pa[1, :N], pd[0, :N], pd[1, :N], r1,
    W2_out, W2_root, b2_out.reshape(1, DOUT))

  (pa2,) = _sc_l2(p, rows2, cols2, zd[:, :DOUT])

  out = pl.pallas_call(
      _phase_e,
      grid=grid,
      in_specs=[rows_spec(DOUT), rows_spec(DOUT), rows_spec(DOUT),
                rows_spec(16), rows_spec(DOUT)],
      out_specs=rows_spec(DOUT),
      out_shape=jax.ShapeDtypeStruct((N, DOUT), jnp.float32),
  )(p, pa2[0, :N], pa2[1, :N], dinv, r2)
  return out
